# baseline (device time: 138549 ns/iter reference)
import jax
import jax.numpy as jnp
from jax import lax
from jax.experimental import pallas as pl
from jax.experimental.pallas import tpu as pltpu

N_DEV = 4
SQ = 1024
HQ = 8
DH = 128
D_MODEL = 1024
SCALE = 0.08838834764831843
BLOCK = 64
NSTRIDE = 4
NREP = 4
GROUP = NREP * BLOCK


def _body(x_ref, wq_ref, k_ref, v_ref, wo_ref, out_ref,
          qp_ref, kg_ref, vg_ref, wob_ref, ctx_ref, own_ref,
          rs_send, rs_recv, ag_send, ag_recv,
          rs_send_sems, rs_recv_sems, ag_send_sems, ag_recv_sems):
    my = lax.axis_index("i")

    barrier_sem = pltpu.get_barrier_semaphore()
    for d in range(1, N_DEV):
        pl.semaphore_signal(barrier_sem, inc=1,
                            device_id=((my + d) % N_DEV,),
                            device_id_type=pl.DeviceIdType.MESH)
    pl.semaphore_wait(barrier_sem, N_DEV - 1)

    q = jnp.dot(x_ref[...].astype(jnp.bfloat16),
                wq_ref[...].astype(jnp.bfloat16),
                preferred_element_type=jnp.float32).astype(jnp.bfloat16)

    for b in range(SQ // BLOCK):
        r, s = b // 4, b % 4
        src = slice(b * BLOCK, (b + 1) * BLOCK)
        dst = slice(s * GROUP + r * BLOCK, s * GROUP + (r + 1) * BLOCK)
        qp_ref[dst, :] = q[src, :]
        kg_ref[dst, :] = k_ref[src, :].astype(jnp.bfloat16)
        vg_ref[dst, :] = v_ref[src, :].astype(jnp.bfloat16)
    wob_ref[...] = wo_ref[...].astype(jnp.bfloat16)

    def finalize(q_):
        for s_ in range(N_DEV):
            if s_ != q_:
                pltpu.make_async_remote_copy(
                    src_ref=rs_send.at[s_], dst_ref=rs_recv.at[s_],
                    send_sem=rs_send_sems.at[s_],
                    recv_sem=rs_recv_sems.at[s_],
                    device_id=(s_,), device_id_type=pl.DeviceIdType.MESH,
                ).wait_recv()
        own = own_ref[...]
        for s_ in range(N_DEV):
            if s_ != q_:
                own = own + rs_recv[s_].astype(jnp.float32)
        for r in range(NREP):
            out_ref[(r * NSTRIDE + q_) * BLOCK:
                    (r * NSTRIDE + q_ + 1) * BLOCK, :] = (
                own[r * BLOCK:(r + 1) * BLOCK, :])
        ag_send[...] = own.astype(jnp.bfloat16)
        for o in range(N_DEV):
            if o != q_:
                pltpu.make_async_remote_copy(
                    src_ref=ag_send, dst_ref=ag_recv.at[q_],
                    send_sem=ag_send_sems.at[o],
                    recv_sem=ag_recv_sems.at[q_],
                    device_id=(o,), device_id_type=pl.DeviceIdType.MESH,
                ).start()

    for s in range(N_DEV):
        rows = slice(s * GROUP, (s + 1) * GROUP)

        for h in range(HQ):
            cols = slice(h * DH, (h + 1) * DH)
            qs = qp_ref[rows, cols]
            sc = lax.dot_general(qs, kg_ref[rows, cols],
                                 (((1,), (1,)), ((), ())),
                                 preferred_element_type=jnp.float32)
            w = jnp.exp(sc * SCALE)
            rsum = 1.0 / jnp.sum(w, axis=-1, keepdims=True)
            ctx = jnp.dot(w.astype(jnp.bfloat16), vg_ref[rows, cols],
                          preferred_element_type=jnp.float32)
            ctx_ref[:, cols] = (ctx * rsum).astype(jnp.bfloat16)

        if s > 0:
            @pl.when(my == s - 1)
            def _():
                finalize(s - 1)

        partial = jnp.dot(ctx_ref[...], wob_ref[...],
                          preferred_element_type=jnp.float32)

        @pl.when(my == s)
        def _():
            own_ref[...] = partial

        @pl.when(my != s)
        def _():
            rs_send[s] = partial.astype(jnp.bfloat16)
            pltpu.make_async_remote_copy(
                src_ref=rs_send.at[s], dst_ref=rs_recv.at[my],
                send_sem=rs_send_sems.at[s], recv_sem=rs_recv_sems.at[my],
                device_id=(s,), device_id_type=pl.DeviceIdType.MESH,
            ).start()

    @pl.when(my == N_DEV - 1)
    def _():
        finalize(N_DEV - 1)

    for s in range(N_DEV):
        @pl.when(my != s)
        def _():
            pltpu.make_async_remote_copy(
                src_ref=rs_send.at[s], dst_ref=rs_recv.at[my],
                send_sem=rs_send_sems.at[s], recv_sem=rs_recv_sems.at[my],
                device_id=(s,), device_id_type=pl.DeviceIdType.MESH,
            ).wait_send()

    for o in range(N_DEV):
        @pl.when(my != o)
        def _():
            pltpu.make_async_remote_copy(
                src_ref=ag_send, dst_ref=ag_recv.at[o],
                send_sem=ag_send_sems.at[o], recv_sem=ag_recv_sems.at[o],
                device_id=(o,), device_id_type=pl.DeviceIdType.MESH,
            ).wait_recv()
        for r in range(NREP):
            @pl.when(my != o)
            def _():
                out_ref[(r * NSTRIDE + o) * BLOCK:
                        (r * NSTRIDE + o + 1) * BLOCK, :] = (
                    ag_recv[o, r * BLOCK:(r + 1) * BLOCK, :]
                    .astype(jnp.float32))

    for o in range(N_DEV):
        @pl.when(my != o)
        def _():
            pltpu.make_async_remote_copy(
                src_ref=ag_send, dst_ref=ag_recv.at[my],
                send_sem=ag_send_sems.at[o], recv_sem=ag_recv_sems.at[my],
                device_id=(o,), device_id_type=pl.DeviceIdType.MESH,
            ).wait_send()


def kernel(x, Wq, K_ext, V_ext, Wo):
    my = lax.axis_index("i")
    k = lax.dynamic_slice_in_dim(
        K_ext[0].reshape(SQ, 32 * DH), my * HQ * DH, HQ * DH, axis=1)
    v = lax.dynamic_slice_in_dim(
        V_ext[0].reshape(SQ, 32 * DH), my * HQ * DH, HQ * DH, axis=1)

    out = pl.pallas_call(
        _body,
        out_shape=jax.ShapeDtypeStruct((SQ, D_MODEL), jnp.float32),
        in_specs=[pl.BlockSpec(memory_space=pltpu.VMEM)] * 5,
        out_specs=pl.BlockSpec(memory_space=pltpu.VMEM),
        scratch_shapes=[
            pltpu.VMEM((SQ, HQ * DH), jnp.bfloat16),
            pltpu.VMEM((SQ, HQ * DH), jnp.bfloat16),
            pltpu.VMEM((SQ, HQ * DH), jnp.bfloat16),
            pltpu.VMEM((HQ * DH, D_MODEL), jnp.bfloat16),
            pltpu.VMEM((GROUP, HQ * DH), jnp.bfloat16),
            pltpu.VMEM((GROUP, D_MODEL), jnp.float32),
            pltpu.VMEM((N_DEV, GROUP, D_MODEL), jnp.bfloat16),
            pltpu.VMEM((N_DEV, GROUP, D_MODEL), jnp.bfloat16),
            pltpu.VMEM((GROUP, D_MODEL), jnp.bfloat16),
            pltpu.VMEM((N_DEV, GROUP, D_MODEL), jnp.bfloat16),
            pltpu.SemaphoreType.DMA((N_DEV,)),
            pltpu.SemaphoreType.DMA((N_DEV,)),
            pltpu.SemaphoreType.DMA((N_DEV,)),
            pltpu.SemaphoreType.DMA((N_DEV,)),
        ],
        compiler_params=pltpu.CompilerParams(
            collective_id=0,
            vmem_limit_bytes=120 * 1024 * 1024,
        ),
    )(x[0], Wq, k, v, Wo)
    return out[None]


# device time: 59354 ns/iter; 2.3343x vs baseline; 2.3343x over previous
import jax
import jax.numpy as jnp
from jax import lax
from jax.experimental import pallas as pl
from jax.experimental.pallas import tpu as pltpu

N_DEV = 4
SQ = 1024
HQ = 8
DH = 128
D_MODEL = 1024
SCALE = 0.08838834764831843
BLOCK = 64
NSTRIDE = 4
NREP = 4
GROUP = NREP * BLOCK


def _body(x_ref, wq_ref, k_ref, v_ref, wo_ref, out_ref,
          qp_ref, kg_ref, vg_ref, wob_ref, ctx_ref, own_ref,
          rs_send, rs_recv, ag_send, ag_recv,
          rs_send_sems, rs_recv_sems, ag_send_sems, ag_recv_sems):
    my = lax.axis_index("i")

    barrier_sem = pltpu.get_barrier_semaphore()
    for d in range(1, N_DEV):
        pl.semaphore_signal(barrier_sem, inc=1,
                            device_id=((my + d) % N_DEV,),
                            device_id_type=pl.DeviceIdType.MESH)
    pl.semaphore_wait(barrier_sem, N_DEV - 1)

    q = jnp.dot(x_ref[...].astype(jnp.bfloat16),
                wq_ref[...].astype(jnp.bfloat16),
                preferred_element_type=jnp.float32).astype(jnp.bfloat16)

    for b in range(SQ // BLOCK):
        r, s = b // 4, b % 4
        src = slice(b * BLOCK, (b + 1) * BLOCK)
        dst = slice(s * GROUP + r * BLOCK, s * GROUP + (r + 1) * BLOCK)
        qp_ref[dst, :] = q[src, :]
        kg_ref[dst, :] = k_ref[src, :]
        vg_ref[dst, :] = v_ref[src, :]
    wob_ref[...] = wo_ref[...].astype(jnp.bfloat16)

    for s in range(N_DEV):
        rows = slice(s * GROUP, (s + 1) * GROUP)

        for h in range(HQ):
            cols = slice(h * DH, (h + 1) * DH)
            qs = qp_ref[rows, cols]
            sc = lax.dot_general(qs, kg_ref[rows, cols],
                                 (((1,), (1,)), ((), ())),
                                 preferred_element_type=jnp.float32)
            w = jnp.exp(sc * SCALE)
            rsum = 1.0 / jnp.sum(w, axis=-1, keepdims=True)
            ctx = jnp.dot(w.astype(jnp.bfloat16), vg_ref[rows, cols],
                          preferred_element_type=jnp.float32)
            ctx_ref[:, cols] = (ctx * rsum).astype(jnp.bfloat16)

        partial = jnp.dot(ctx_ref[...], wob_ref[...],
                          preferred_element_type=jnp.float32)

        @pl.when(my == s)
        def _():
            own_ref[...] = partial

        @pl.when(my != s)
        def _():
            rs_send[s] = partial.astype(jnp.bfloat16)
            pltpu.make_async_remote_copy(
                src_ref=rs_send.at[s], dst_ref=rs_recv.at[my],
                send_sem=rs_send_sems.at[s], recv_sem=rs_recv_sems.at[my],
                device_id=(s,), device_id_type=pl.DeviceIdType.MESH,
            ).start()

    for q_ in range(N_DEV):
        @pl.when(my == q_)
        def _():
            for s_ in range(N_DEV):
                if s_ != q_:
                    pltpu.make_async_remote_copy(
                        src_ref=rs_send.at[s_], dst_ref=rs_recv.at[s_],
                        send_sem=rs_send_sems.at[s_],
                        recv_sem=rs_recv_sems.at[s_],
                        device_id=(s_,),
                        device_id_type=pl.DeviceIdType.MESH,
                    ).wait_recv()
            own = own_ref[...]
            for s_ in range(N_DEV):
                if s_ != q_:
                    own = own + rs_recv[s_].astype(jnp.float32)
            for r in range(NREP):
                out_ref[(r * NSTRIDE + q_) * BLOCK:
                        (r * NSTRIDE + q_ + 1) * BLOCK, :] = (
                    own[r * BLOCK:(r + 1) * BLOCK, :])
            ag_send[...] = own.astype(jnp.bfloat16)
            for o in range(N_DEV):
                if o != q_:
                    pltpu.make_async_remote_copy(
                        src_ref=ag_send, dst_ref=ag_recv.at[q_],
                        send_sem=ag_send_sems.at[o],
                        recv_sem=ag_recv_sems.at[q_],
                        device_id=(o,),
                        device_id_type=pl.DeviceIdType.MESH,
                    ).start()

    for s in range(N_DEV):
        @pl.when(my != s)
        def _():
            pltpu.make_async_remote_copy(
                src_ref=rs_send.at[s], dst_ref=rs_recv.at[my],
                send_sem=rs_send_sems.at[s], recv_sem=rs_recv_sems.at[my],
                device_id=(s,), device_id_type=pl.DeviceIdType.MESH,
            ).wait_send()

    for o in range(N_DEV):
        @pl.when(my != o)
        def _():
            pltpu.make_async_remote_copy(
                src_ref=ag_send, dst_ref=ag_recv.at[o],
                send_sem=ag_send_sems.at[o], recv_sem=ag_recv_sems.at[o],
                device_id=(o,), device_id_type=pl.DeviceIdType.MESH,
            ).wait_recv()
        for r in range(NREP):
            @pl.when(my != o)
            def _():
                out_ref[(r * NSTRIDE + o) * BLOCK:
                        (r * NSTRIDE + o + 1) * BLOCK, :] = (
                    ag_recv[o, r * BLOCK:(r + 1) * BLOCK, :]
                    .astype(jnp.float32))

    for o in range(N_DEV):
        @pl.when(my != o)
        def _():
            pltpu.make_async_remote_copy(
                src_ref=ag_send, dst_ref=ag_recv.at[my],
                send_sem=ag_send_sems.at[o], recv_sem=ag_recv_sems.at[my],
                device_id=(o,), device_id_type=pl.DeviceIdType.MESH,
            ).wait_send()


def kernel(x, Wq, K_ext, V_ext, Wo):
    my = lax.axis_index("i")

    def heads(t):
        g = lax.dynamic_slice_in_dim(t[0], my * HQ, HQ, axis=1)
        return g.astype(jnp.bfloat16).reshape(SQ, HQ * DH)

    k = heads(K_ext)
    v = heads(V_ext)

    out = pl.pallas_call(
        _body,
        out_shape=jax.ShapeDtypeStruct((SQ, D_MODEL), jnp.float32),
        in_specs=[pl.BlockSpec(memory_space=pltpu.VMEM)] * 5,
        out_specs=pl.BlockSpec(memory_space=pltpu.VMEM),
        scratch_shapes=[
            pltpu.VMEM((SQ, HQ * DH), jnp.bfloat16),
            pltpu.VMEM((SQ, HQ * DH), jnp.bfloat16),
            pltpu.VMEM((SQ, HQ * DH), jnp.bfloat16),
            pltpu.VMEM((HQ * DH, D_MODEL), jnp.bfloat16),
            pltpu.VMEM((GROUP, HQ * DH), jnp.bfloat16),
            pltpu.VMEM((GROUP, D_MODEL), jnp.float32),
            pltpu.VMEM((N_DEV, GROUP, D_MODEL), jnp.bfloat16),
            pltpu.VMEM((N_DEV, GROUP, D_MODEL), jnp.bfloat16),
            pltpu.VMEM((GROUP, D_MODEL), jnp.bfloat16),
            pltpu.VMEM((N_DEV, GROUP, D_MODEL), jnp.bfloat16),
            pltpu.SemaphoreType.DMA((N_DEV,)),
            pltpu.SemaphoreType.DMA((N_DEV,)),
            pltpu.SemaphoreType.DMA((N_DEV,)),
            pltpu.SemaphoreType.DMA((N_DEV,)),
        ],
        compiler_params=pltpu.CompilerParams(
            collective_id=0,
            vmem_limit_bytes=120 * 1024 * 1024,
        ),
    )(x[0], Wq, k, v, Wo)
    return out[None]


# device time: 58562 ns/iter; 2.3659x vs baseline; 1.0135x over previous
import jax
import jax.numpy as jnp
from jax import lax
from jax.experimental import pallas as pl
from jax.experimental.pallas import tpu as pltpu

N_DEV = 4
SQ = 1024
HQ = 8
DH = 128
D_MODEL = 1024
SCALE = 0.08838834764831843
BLOCK = 64
NSTRIDE = 4
NREP = 4
GROUP = NREP * BLOCK


def _body(x_ref, wq_ref, k_ref, v_ref, wo_ref, out_ref,
          qp_ref, kg_ref, vg_ref, wob_ref, ctx_ref, own_ref,
          rs_send, rs_recv, ag_send, ag_recv,
          rs_send_sems, rs_recv_sems, ag_send_sems, ag_recv_sems):
    my = lax.axis_index("i")

    barrier_sem = pltpu.get_barrier_semaphore()
    for d in range(1, N_DEV):
        pl.semaphore_signal(barrier_sem, inc=1,
                            device_id=((my + d) % N_DEV,),
                            device_id_type=pl.DeviceIdType.MESH)
    pl.semaphore_wait(barrier_sem, N_DEV - 1)

    q = jnp.dot(x_ref[...].astype(jnp.bfloat16),
                wq_ref[...].astype(jnp.bfloat16),
                preferred_element_type=jnp.float32).astype(jnp.bfloat16)

    for b in range(SQ // BLOCK):
        r, s = b // 4, b % 4
        src = slice(b * BLOCK, (b + 1) * BLOCK)
        dst = slice(s * GROUP + r * BLOCK, s * GROUP + (r + 1) * BLOCK)
        qp_ref[dst, :] = q[src, :]
        kg_ref[dst, :] = k_ref[src, :]
        vg_ref[dst, :] = v_ref[src, :]
    wob_ref[...] = wo_ref[...].astype(jnp.bfloat16)

    def class_attn(s):
        rows = slice(s * GROUP, (s + 1) * GROUP)
        for h in range(HQ):
            cols = slice(h * DH, (h + 1) * DH)
            qs = qp_ref[rows, cols]
            sc = lax.dot_general(qs, kg_ref[rows, cols],
                                 (((1,), (1,)), ((), ())),
                                 preferred_element_type=jnp.float32)
            w = jnp.exp(sc * SCALE)
            rsum = 1.0 / jnp.sum(w, axis=-1, keepdims=True)
            ctx = jnp.dot(w.astype(jnp.bfloat16), vg_ref[rows, cols],
                          preferred_element_type=jnp.float32)
            ctx_ref[:, cols] = (ctx * rsum).astype(jnp.bfloat16)
        return jnp.dot(ctx_ref[...], wob_ref[...],
                       preferred_element_type=jnp.float32)

    for s in range(N_DEV):
        @pl.when(my != s)
        def _():
            partial = class_attn(s)
            rs_send[s] = partial.astype(jnp.bfloat16)
            pltpu.make_async_remote_copy(
                src_ref=rs_send.at[s], dst_ref=rs_recv.at[my],
                send_sem=rs_send_sems.at[s], recv_sem=rs_recv_sems.at[my],
                device_id=(s,), device_id_type=pl.DeviceIdType.MESH,
            ).start()

    for s in range(N_DEV):
        @pl.when(my == s)
        def _():
            own_ref[...] = class_attn(s)

    for q_ in range(N_DEV):
        @pl.when(my == q_)
        def _():
            for s_ in range(N_DEV):
                if s_ != q_:
                    pltpu.make_async_remote_copy(
                        src_ref=rs_send.at[s_], dst_ref=rs_recv.at[s_],
                        send_sem=rs_send_sems.at[s_],
                        recv_sem=rs_recv_sems.at[s_],
                        device_id=(s_,),
                        device_id_type=pl.DeviceIdType.MESH,
                    ).wait_recv()
            own = own_ref[...]
            for s_ in range(N_DEV):
                if s_ != q_:
                    own = own + rs_recv[s_].astype(jnp.float32)
            for r in range(NREP):
                out_ref[(r * NSTRIDE + q_) * BLOCK:
                        (r * NSTRIDE + q_ + 1) * BLOCK, :] = (
                    own[r * BLOCK:(r + 1) * BLOCK, :])
            ag_send[...] = own.astype(jnp.bfloat16)
            for o in range(N_DEV):
                if o != q_:
                    pltpu.make_async_remote_copy(
                        src_ref=ag_send, dst_ref=ag_recv.at[q_],
                        send_sem=ag_send_sems.at[o],
                        recv_sem=ag_recv_sems.at[q_],
                        device_id=(o,),
                        device_id_type=pl.DeviceIdType.MESH,
                    ).start()

    for s in range(N_DEV):
        @pl.when(my != s)
        def _():
            pltpu.make_async_remote_copy(
                src_ref=rs_send.at[s], dst_ref=rs_recv.at[my],
                send_sem=rs_send_sems.at[s], recv_sem=rs_recv_sems.at[my],
                device_id=(s,), device_id_type=pl.DeviceIdType.MESH,
            ).wait_send()

    for o in range(N_DEV):
        @pl.when(my != o)
        def _():
            pltpu.make_async_remote_copy(
                src_ref=ag_send, dst_ref=ag_recv.at[o],
                send_sem=ag_send_sems.at[o], recv_sem=ag_recv_sems.at[o],
                device_id=(o,), device_id_type=pl.DeviceIdType.MESH,
            ).wait_recv()
        for r in range(NREP):
            @pl.when(my != o)
            def _():
                out_ref[(r * NSTRIDE + o) * BLOCK:
                        (r * NSTRIDE + o + 1) * BLOCK, :] = (
                    ag_recv[o, r * BLOCK:(r + 1) * BLOCK, :]
                    .astype(jnp.float32))

    for o in range(N_DEV):
        @pl.when(my != o)
        def _():
            pltpu.make_async_remote_copy(
                src_ref=ag_send, dst_ref=ag_recv.at[my],
                send_sem=ag_send_sems.at[o], recv_sem=ag_recv_sems.at[my],
                device_id=(o,), device_id_type=pl.DeviceIdType.MESH,
            ).wait_send()


def kernel(x, Wq, K_ext, V_ext, Wo):
    my = lax.axis_index("i")

    def heads(t):
        g = lax.dynamic_slice_in_dim(t[0], my * HQ, HQ, axis=1)
        return g.astype(jnp.bfloat16).reshape(SQ, HQ * DH)

    k = heads(K_ext)
    v = heads(V_ext)

    out = pl.pallas_call(
        _body,
        out_shape=jax.ShapeDtypeStruct((SQ, D_MODEL), jnp.float32),
        in_specs=[pl.BlockSpec(memory_space=pltpu.VMEM)] * 5,
        out_specs=pl.BlockSpec(memory_space=pltpu.VMEM),
        scratch_shapes=[
            pltpu.VMEM((SQ, HQ * DH), jnp.bfloat16),
            pltpu.VMEM((SQ, HQ * DH), jnp.bfloat16),
            pltpu.VMEM((SQ, HQ * DH), jnp.bfloat16),
            pltpu.VMEM((HQ * DH, D_MODEL), jnp.bfloat16),
            pltpu.VMEM((GROUP, HQ * DH), jnp.bfloat16),
            pltpu.VMEM((GROUP, D_MODEL), jnp.float32),
            pltpu.VMEM((N_DEV, GROUP, D_MODEL), jnp.bfloat16),
            pltpu.VMEM((N_DEV, GROUP, D_MODEL), jnp.bfloat16),
            pltpu.VMEM((GROUP, D_MODEL), jnp.bfloat16),
            pltpu.VMEM((N_DEV, GROUP, D_MODEL), jnp.bfloat16),
            pltpu.SemaphoreType.DMA((N_DEV,)),
            pltpu.SemaphoreType.DMA((N_DEV,)),
            pltpu.SemaphoreType.DMA((N_DEV,)),
            pltpu.SemaphoreType.DMA((N_DEV,)),
        ],
        compiler_params=pltpu.CompilerParams(
            collective_id=0,
            vmem_limit_bytes=120 * 1024 * 1024,
        ),
    )(x[0], Wq, k, v, Wo)
    return out[None]
